# Initial kernel scaffold; baseline (speedup 1.0000x reference)
import jax
import jax.numpy as jnp
from jax.experimental import pallas as pl


def kernel(x, emb):
    raise NotImplementedError("write your pallas kernel here")


# speedup vs baseline: 1.0243x; 1.0243x over previous
import functools

import jax
import jax.numpy as jnp
from jax import lax
from jax.experimental import pallas as pl
from jax.experimental.pallas import tpu as pltpu
from jax.experimental.pallas import tpu_sc as plsc

DIM = 2048
SEQ_LEN = 8192
NUM_CORES = 2
NUM_SUBCORES = 16
LANES = 16
NUM_WORKERS = NUM_CORES * NUM_SUBCORES
ROWS_PER_WORKER = SEQ_LEN // NUM_WORKERS
CHUNK_ROWS = 32
NUM_CHUNKS = ROWS_PER_WORKER // CHUNK_ROWS
VECS_PER_ROW = DIM // LANES


def _scale_chunk(buf, scale):
    def row_body(i, _):
        for j in range(VECS_PER_ROW):
            sl = pl.ds(j * LANES, LANES)
            buf[i, sl] = buf[i, sl] * scale
        return 0

    lax.fori_loop(0, CHUNK_ROWS, row_body, 0)


@functools.partial(
    pl.kernel,
    out_type=jax.ShapeDtypeStruct((SEQ_LEN, DIM), jnp.float32),
    mesh=plsc.VectorSubcoreMesh(core_axis_name="c", subcore_axis_name="s"),
    scratch_types=[pltpu.VMEM((CHUNK_ROWS, DIM), jnp.float32)],
)
def _pos_emb_sc(emb_hbm, out_hbm, buf):
    scale = jnp.float32(DIM ** -0.5)
    wid = lax.axis_index("s") * NUM_CORES + lax.axis_index("c")
    base = wid * ROWS_PER_WORKER
    for k in range(NUM_CHUNKS):
        row0 = base + k * CHUNK_ROWS
        pltpu.sync_copy(emb_hbm.at[pl.ds(row0, CHUNK_ROWS)], buf)
        _scale_chunk(buf, scale)
        pltpu.sync_copy(buf, out_hbm.at[pl.ds(row0, CHUNK_ROWS)])


def kernel(x, emb):
    seq_len = x.shape[1]
    assert seq_len == SEQ_LEN
    return _pos_emb_sc(emb)
